# Initial kernel scaffold; baseline (speedup 1.0000x reference)
import jax
import jax.numpy as jnp
from jax.experimental import pallas as pl


def kernel(x, W_router, gate_w, up_w, down_w):
    raise NotImplementedError("write your pallas kernel here")


# speedup vs baseline: 1.3429x; 1.3429x over previous
import functools

import jax
import jax.numpy as jnp
from jax import lax
from jax.experimental import pallas as pl
from jax.experimental.pallas import tpu as pltpu
from jax.experimental.pallas import tpu_sc as plsc

D_MODEL = 1024
N_EXPERTS = 8
N_ACTIVE = 2
CAPACITY_FACTOR = 1.25
D_FF = 4096
AUX_COEFF = 0.01

SC_CORES = 2
SC_SUBCORES = 16
SC_WORKERS = SC_CORES * SC_SUBCORES
ROW_CHUNK = 64

F_BLK = 1024


def _router_body(cap, n_tok, x_ref, wr_ref, dst_ref, gidx_ref, aw_ref, aux_ref):
    n_flat = N_ACTIVE * n_tok
    x = x_ref[...]
    wr = wr_ref[...]
    logits = lax.dot_general(
        wr, x, (((1,), (1,)), ((), ())),
        preferred_element_type=jnp.float32,
    )
    m = jnp.max(logits, axis=0, keepdims=True)
    e = jnp.exp(logits - m)
    probs = e / jnp.sum(e, axis=0, keepdims=True)

    iota_e = lax.broadcasted_iota(jnp.int32, (N_EXPERTS, n_tok), 0)
    p1 = jnp.max(probs, axis=0, keepdims=True)
    i1 = jnp.min(jnp.where(probs == p1, iota_e, N_EXPERTS), axis=0, keepdims=True)
    probs2 = jnp.where(iota_e == i1, -1.0, probs)
    p2 = jnp.max(probs2, axis=0, keepdims=True)
    i2 = jnp.min(jnp.where(probs2 == p2, iota_e, N_EXPERTS), axis=0, keepdims=True)
    sw = p1 + p2 + 1e-9
    w0 = p1 / sw
    w1 = p2 / sw

    fe = jnp.concatenate([i1, i2], axis=1)
    oh = (fe == lax.broadcasted_iota(jnp.int32, (N_EXPERTS, n_flat), 0)).astype(
        jnp.float32)
    a = oh
    s = 1
    while s < n_flat:
        a = a + jnp.concatenate(
            [jnp.zeros((N_EXPERTS, s), jnp.float32), a[:, :-s]], axis=1)
        s *= 2
    excl = a - oh
    rank = jnp.sum(oh * excl, axis=0, keepdims=True).astype(jnp.int32)
    acc = rank < cap

    flat_w = jnp.concatenate([w0, w1], axis=1)
    trash = N_EXPERTS * cap
    dst_ref[...] = jnp.where(acc, fe * cap + rank, trash)
    gidx_ref[...] = fe * cap + jnp.minimum(rank, cap - 1)
    aw_ref[...] = jnp.where(acc, flat_w, 0.0)

    tot_e = a[:, n_flat - 1:n_flat]
    cnt = jnp.minimum(tot_e, float(cap))
    total = jnp.maximum(jnp.sum(cnt), 1.0)
    f_i = cnt / total
    p_mean = jnp.sum(probs, axis=1, keepdims=True) / float(n_tok)
    aux = AUX_COEFF * N_EXPERTS * jnp.sum(f_i * p_mean)
    aux_ref[...] = jnp.reshape(aux, (1, 1))


def _ffn_body(x_ref, gw_ref, uw_ref, dw_ref, y_ref):
    j = pl.program_id(1)
    xb = x_ref[...].astype(jnp.bfloat16)
    gw = gw_ref[0].astype(jnp.bfloat16)
    uw = uw_ref[0].astype(jnp.bfloat16)
    dw = dw_ref[0].astype(jnp.bfloat16)
    dn = (((1,), (1,)), ((), ()))
    g = lax.dot_general(xb, gw, dn, preferred_element_type=jnp.float32)
    u = lax.dot_general(xb, uw, dn, preferred_element_type=jnp.float32)
    h = (g * (1.0 / (1.0 + jnp.exp(-g)))) * u
    y = lax.dot_general(h.astype(jnp.bfloat16), dw, dn,
                        preferred_element_type=jnp.float32)

    @pl.when(j == 0)
    def _():
        y_ref[0] = y

    @pl.when(j != 0)
    def _():
        y_ref[0] = y_ref[0] + y


def _final_body(y0_ref, y1_ref, aw0_ref, aw1_ref, o_ref):
    aw0 = aw0_ref[...]
    aw1 = aw1_ref[...]
    o_ref[...] = (jnp.where(aw0 > 0, aw0 * y0_ref[...], 0.0)
                  + jnp.where(aw1 > 0, aw1 * y1_ref[...], 0.0))


def _sc_mesh():
    return plsc.VectorSubcoreMesh(
        core_axis_name="c", subcore_axis_name="s",
        num_cores=SC_CORES, num_subcores=SC_SUBCORES)


def _make_dispatch(n_tok, n_rows):
    n_flat = N_ACTIVE * n_tok
    per_w = n_flat // SC_WORKERS
    n_ch = per_w // ROW_CHUNK

    @functools.partial(
        pl.kernel,
        out_type=jax.ShapeDtypeStruct((n_rows, D_MODEL), jnp.float32),
        mesh=_sc_mesh(),
        scratch_types=[
            pltpu.VMEM((ROW_CHUNK,), jnp.int32),
            pltpu.VMEM((ROW_CHUNK, D_MODEL), jnp.float32),
            pltpu.SemaphoreType.DMA,
        ],
    )
    def dispatch(x_hbm, dst_hbm, out_hbm, idx_v, rows_v, sem):
        wid = lax.axis_index("s") * SC_CORES + lax.axis_index("c")
        base = wid * per_w

        def body(i, carry):
            off = base + i * ROW_CHUNK
            src = lax.rem(off, n_tok)
            pltpu.sync_copy(dst_hbm.at[pl.ds(off, ROW_CHUNK)], idx_v)
            pltpu.sync_copy(x_hbm.at[pl.ds(src, ROW_CHUNK)], rows_v)
            pltpu.async_copy(rows_v, out_hbm.at[idx_v], sem).wait()
            return carry

        lax.fori_loop(0, n_ch, body, 0)

    return dispatch


def _make_combine(n_tok, n_rows):
    n_flat = N_ACTIVE * n_tok
    per_w = n_flat // SC_WORKERS
    n_ch = per_w // ROW_CHUNK

    @functools.partial(
        pl.kernel,
        out_type=jax.ShapeDtypeStruct((n_flat, D_MODEL), jnp.float32),
        mesh=_sc_mesh(),
        scratch_types=[
            pltpu.VMEM((ROW_CHUNK,), jnp.int32),
            pltpu.VMEM((ROW_CHUNK, D_MODEL), jnp.float32),
            pltpu.SemaphoreType.DMA,
        ],
    )
    def combine(y_hbm, gidx_hbm, out_hbm, idx_v, rows_v, sem):
        wid = lax.axis_index("s") * SC_CORES + lax.axis_index("c")
        base = wid * per_w

        def body(i, carry):
            off = base + i * ROW_CHUNK
            pltpu.sync_copy(gidx_hbm.at[pl.ds(off, ROW_CHUNK)], idx_v)
            pltpu.async_copy(y_hbm.at[idx_v], rows_v, sem).wait()
            pltpu.sync_copy(rows_v, out_hbm.at[pl.ds(off, ROW_CHUNK)])
            return carry

        lax.fori_loop(0, n_ch, body, 0)

    return combine


def kernel(x, W_router, gate_w, up_w, down_w):
    bb, tt, d = x.shape
    n_tok = bb * tt
    n_flat = N_ACTIVE * n_tok
    cap = max(int(tt * N_ACTIVE * CAPACITY_FACTOR / N_EXPERTS), 1)
    n_rows = N_EXPERTS * cap + 8

    x_flat = x.reshape(n_tok, d)

    dst, gidx, aw, aux = pl.pallas_call(
        functools.partial(_router_body, cap, n_tok),
        out_shape=[
            jax.ShapeDtypeStruct((1, n_flat), jnp.int32),
            jax.ShapeDtypeStruct((1, n_flat), jnp.int32),
            jax.ShapeDtypeStruct((1, n_flat), jnp.float32),
            jax.ShapeDtypeStruct((1, 1), jnp.float32),
        ],
    )(x_flat, W_router)

    buf = _make_dispatch(n_tok, n_rows)(x_flat, dst.reshape(n_flat))

    grid = (N_EXPERTS, D_FF // F_BLK)
    y = pl.pallas_call(
        _ffn_body,
        grid=grid,
        in_specs=[
            pl.BlockSpec((cap, d), lambda e, j: (e, 0)),
            pl.BlockSpec((1, F_BLK, d), lambda e, j: (e, j, 0)),
            pl.BlockSpec((1, F_BLK, d), lambda e, j: (e, j, 0)),
            pl.BlockSpec((1, d, F_BLK), lambda e, j: (e, 0, j)),
        ],
        out_specs=pl.BlockSpec((1, cap, d), lambda e, j: (e, 0, 0)),
        out_shape=jax.ShapeDtypeStruct((N_EXPERTS, cap, d), jnp.float32),
        compiler_params=pltpu.CompilerParams(
            dimension_semantics=("parallel", "arbitrary")),
    )(buf, gate_w, up_w, down_w)

    yg = _make_combine(n_tok, N_EXPERTS * cap)(
        y.reshape(N_EXPERTS * cap, d), gidx.reshape(n_flat))

    aw_col = aw.reshape(n_flat, 1)
    blk = 512
    out_flat = pl.pallas_call(
        _final_body,
        grid=(n_tok // blk,),
        in_specs=[
            pl.BlockSpec((blk, d), lambda i: (i, 0)),
            pl.BlockSpec((blk, d), lambda i, n=n_tok // blk: (i + n, 0)),
            pl.BlockSpec((blk, 1), lambda i: (i, 0)),
            pl.BlockSpec((blk, 1), lambda i, n=n_tok // blk: (i + n, 0)),
        ],
        out_specs=pl.BlockSpec((blk, d), lambda i: (i, 0)),
        out_shape=jax.ShapeDtypeStruct((n_tok, d), jnp.float32),
        compiler_params=pltpu.CompilerParams(
            dimension_semantics=("parallel",)),
    )(yg, yg, aw_col, aw_col)

    return out_flat.reshape(bb, tt, d), aux.reshape(())
